# baseline (speedup 1.0000x reference)
import jax
import jax.numpy as jnp
from jax.experimental import pallas as pl
from jax.experimental.pallas import tpu as pltpu

NO_ELECTRON_WEIGHT = 0.1
SALIENCE_ALPHA = 0.25
SALIENCE_GAMMA = 2.0

B = 4
P = 4096
N = 512
CHUNK = 1024
NC = P // CHUNK
ROWS = CHUNK // 8

MASK_ELEMS = float(B * P * N)
SAL_ELEMS = float(B * 65536)
DICE_SLOTS = float(B * N)
NQ = 2048.0


def _bce(logits, targets):
    return (jnp.maximum(logits, 0.0) - logits * targets
            + jnp.log1p(jnp.exp(-jnp.abs(logits))))


def _loss_body(pred_ref, lab_ref, mask_ref, true_ref, pos_ref, post_ref,
               sal_ref, salt_ref, out_ref, acc_ref, bce_ref, spt_ref,
               sp_ref, st_ref):
    b = pl.program_id(0)
    c = pl.program_id(1)
    first = jnp.logical_and(b == 0, c == 0)

    @pl.when(first)
    def _init():
        acc_ref[0] = 0.0
        bce_ref[...] = jnp.zeros((8, N), jnp.float32)

    zero = jnp.zeros((8, N), jnp.float32)
    abce, apt, ap, at = zero, zero, zero, zero
    for i in range(ROWS):
        x = mask_ref[0, i * 8:(i + 1) * 8, :]
        t = true_ref[0, i * 8:(i + 1) * 8, :]
        e = jnp.exp(-x)
        p = 1.0 / (1.0 + e)
        pm = jnp.maximum(p, 1.0 - p)
        abce = abce + (jnp.maximum(x, 0.0) - x * t - jnp.log(pm))
        apt = apt + p * t
        ap = ap + p
        at = at + t

    bce_ref[...] += abce

    @pl.when(c == 0)
    def _dice_init():
        spt_ref[...] = apt
        sp_ref[...] = ap
        st_ref[...] = at

    @pl.when(c > 0)
    def _dice_acc():
        spt_ref[...] += apt
        sp_ref[...] += ap
        st_ref[...] += at

    @pl.when(c == NC - 1)
    def _dice_done():
        num = jnp.sum(spt_ref[...], axis=0, keepdims=True)
        den = (jnp.sum(sp_ref[...], axis=0, keepdims=True)
               + jnp.sum(st_ref[...], axis=0, keepdims=True))
        dice = 1.0 - (2.0 * num + 1.0) / (den + 1.0)
        acc_ref[0] += jnp.sum(dice) * (1.0 / DICE_SLOTS)

    @pl.when(c == 0)
    def _salience():
        sacc = jnp.zeros((32, 128), jnp.float32)
        for i in range(16):
            s = sal_ref[0, i * 32:(i + 1) * 32, :]
            tt = salt_ref[0, i * 32:(i + 1) * 32, :]
            es = jnp.exp(-s)
            p = 1.0 / (1.0 + es)
            pms = jnp.maximum(p, 1.0 - p)
            ce = jnp.maximum(s, 0.0) - s * tt - jnp.log(pms)
            p_t = p * tt + (1.0 - p) * (1.0 - tt)
            om = 1.0 - p_t
            alpha_t = SALIENCE_ALPHA * tt + (1.0 - SALIENCE_ALPHA) * (1.0 - tt)
            sacc = sacc + alpha_t * ce * om * om
        acc_ref[0] += jnp.sum(sacc) * (1.0 / SAL_ELEMS)

    @pl.when(first)
    def _small():
        lab = lab_ref[...].astype(jnp.float32)
        w = jnp.where(lab == 1.0, 1.0, NO_ELECTRON_WEIGHT)
        per_q = _bce(pred_ref[...], lab)
        acc_ref[0] += jnp.sum(w * per_q) / jnp.sum(w)

        d = pos_ref[...] - post_ref[...]
        a = jnp.abs(d)
        h = jnp.where(a < 1.0, 0.5 * d * d, a - 0.5)
        acc_ref[0] += jnp.sum(h) * (1.0 / NQ)

    @pl.when(jnp.logical_and(b == B - 1, c == NC - 1))
    def _emit():
        total = acc_ref[0] + jnp.sum(bce_ref[...]) * (1.0 / MASK_ELEMS)
        out_ref[...] = jnp.broadcast_to(total, (1, 1))


@jax.jit
def kernel(pred_logits, labels, mask_logits, true_masks, pred_positions,
           true_positions, salience_logits, salience_targets):
    pred2 = pred_logits.reshape(16, 128)
    lab2 = labels.reshape(16, 128)
    posp = pred_positions.reshape(32, 128)
    post = true_positions.reshape(32, 128)
    sal3 = salience_logits.reshape(B, 512, 128)
    salt3 = salience_targets.reshape(B, 512, 128)

    grid = (B, NC)
    out = pl.pallas_call(
        _loss_body,
        grid=grid,
        in_specs=[
            pl.BlockSpec((16, 128), lambda b, c: (0, 0)),
            pl.BlockSpec((16, 128), lambda b, c: (0, 0)),
            pl.BlockSpec((1, CHUNK, N), lambda b, c: (b, c, 0)),
            pl.BlockSpec((1, CHUNK, N), lambda b, c: (b, c, 0)),
            pl.BlockSpec((32, 128), lambda b, c: (0, 0)),
            pl.BlockSpec((32, 128), lambda b, c: (0, 0)),
            pl.BlockSpec((1, 512, 128), lambda b, c: (b, 0, 0)),
            pl.BlockSpec((1, 512, 128), lambda b, c: (b, 0, 0)),
        ],
        out_specs=pl.BlockSpec((1, 1), lambda b, c: (0, 0)),
        out_shape=jax.ShapeDtypeStruct((1, 1), jnp.float32),
        scratch_shapes=[
            pltpu.SMEM((1,), jnp.float32),
            pltpu.VMEM((8, N), jnp.float32),
            pltpu.VMEM((8, N), jnp.float32),
            pltpu.VMEM((8, N), jnp.float32),
            pltpu.VMEM((8, N), jnp.float32),
        ],
        compiler_params=pltpu.CompilerParams(
            dimension_semantics=("arbitrary", "arbitrary"),
        ),
    )(pred2, lab2, mask_logits, true_masks, posp, post, sal3, salt3)
    return out.reshape(())


# speedup vs baseline: 1.0866x; 1.0866x over previous
import jax
import jax.numpy as jnp
from jax.experimental import pallas as pl
from jax.experimental.pallas import tpu as pltpu

NO_ELECTRON_WEIGHT = 0.1
SALIENCE_ALPHA = 0.25
SALIENCE_GAMMA = 2.0

B = 4
P = 4096
N = 512
CHUNK = 2048
NC = P // CHUNK
ROWS = CHUNK // 8

MASK_ELEMS = float(B * P * N)
SAL_ELEMS = float(B * 65536)
DICE_SLOTS = float(B * N)
NQ = 2048.0


def _bce(logits, targets):
    return (jnp.maximum(logits, 0.0) - logits * targets
            + jnp.log1p(jnp.exp(-jnp.abs(logits))))


def _loss_body(pred_ref, lab_ref, mask_ref, true_ref, pos_ref, post_ref,
               sal_ref, salt_ref, out_ref, acc_ref, bce_ref, spt_ref,
               sp_ref, st_ref):
    b = pl.program_id(0)
    c = pl.program_id(1)
    first = jnp.logical_and(b == 0, c == 0)

    @pl.when(first)
    def _init():
        acc_ref[0] = 0.0
        bce_ref[...] = jnp.zeros((8, N), jnp.float32)

    zero = jnp.zeros((8, N), jnp.float32)
    abce, apt, ap, at = zero, zero, zero, zero
    GROUP = 32
    for g in range(ROWS // GROUP):
        pprod = jnp.ones((8, N), jnp.float32)
        for k in range(GROUP):
            i = g * GROUP + k
            x = mask_ref[0, i * 8:(i + 1) * 8, :]
            t = true_ref[0, i * 8:(i + 1) * 8, :]
            e = jnp.exp(-x)
            p = 1.0 / (1.0 + e)
            pm = jnp.maximum(p, 1.0 - p)
            pprod = pprod * pm
            abce = abce + (jnp.maximum(x, 0.0) - x * t)
            apt = apt + p * t
            ap = ap + p
            at = at + t
        abce = abce - jnp.log(pprod)

    bce_ref[...] += abce

    @pl.when(c == 0)
    def _dice_init():
        spt_ref[...] = apt
        sp_ref[...] = ap
        st_ref[...] = at

    @pl.when(c > 0)
    def _dice_acc():
        spt_ref[...] += apt
        sp_ref[...] += ap
        st_ref[...] += at

    @pl.when(c == NC - 1)
    def _dice_done():
        num = jnp.sum(spt_ref[...], axis=0, keepdims=True)
        den = (jnp.sum(sp_ref[...], axis=0, keepdims=True)
               + jnp.sum(st_ref[...], axis=0, keepdims=True))
        dice = 1.0 - (2.0 * num + 1.0) / (den + 1.0)
        acc_ref[0] += jnp.sum(dice) * (1.0 / DICE_SLOTS)

    @pl.when(c == 0)
    def _salience():
        sacc = jnp.zeros((32, 128), jnp.float32)
        for i in range(16):
            s = sal_ref[0, i * 32:(i + 1) * 32, :]
            tt = salt_ref[0, i * 32:(i + 1) * 32, :]
            es = jnp.exp(-s)
            p = 1.0 / (1.0 + es)
            pms = jnp.maximum(p, 1.0 - p)
            ce = jnp.maximum(s, 0.0) - s * tt - jnp.log(pms)
            p_t = p * tt + (1.0 - p) * (1.0 - tt)
            om = 1.0 - p_t
            alpha_t = SALIENCE_ALPHA * tt + (1.0 - SALIENCE_ALPHA) * (1.0 - tt)
            sacc = sacc + alpha_t * ce * om * om
        acc_ref[0] += jnp.sum(sacc) * (1.0 / SAL_ELEMS)

    @pl.when(first)
    def _small():
        lab = lab_ref[...].astype(jnp.float32)
        w = jnp.where(lab == 1.0, 1.0, NO_ELECTRON_WEIGHT)
        per_q = _bce(pred_ref[...], lab)
        acc_ref[0] += jnp.sum(w * per_q) / jnp.sum(w)

        d = pos_ref[...] - post_ref[...]
        a = jnp.abs(d)
        h = jnp.where(a < 1.0, 0.5 * d * d, a - 0.5)
        acc_ref[0] += jnp.sum(h) * (1.0 / NQ)

    @pl.when(jnp.logical_and(b == B - 1, c == NC - 1))
    def _emit():
        total = acc_ref[0] + jnp.sum(bce_ref[...]) * (1.0 / MASK_ELEMS)
        out_ref[...] = jnp.broadcast_to(total, (1, 1))


@jax.jit
def kernel(pred_logits, labels, mask_logits, true_masks, pred_positions,
           true_positions, salience_logits, salience_targets):
    pred2 = pred_logits.reshape(16, 128)
    lab2 = labels.reshape(16, 128)
    posp = pred_positions.reshape(32, 128)
    post = true_positions.reshape(32, 128)
    sal3 = salience_logits.reshape(B, 512, 128)
    salt3 = salience_targets.reshape(B, 512, 128)

    grid = (B, NC)
    out = pl.pallas_call(
        _loss_body,
        grid=grid,
        in_specs=[
            pl.BlockSpec((16, 128), lambda b, c: (0, 0)),
            pl.BlockSpec((16, 128), lambda b, c: (0, 0)),
            pl.BlockSpec((1, CHUNK, N), lambda b, c: (b, c, 0)),
            pl.BlockSpec((1, CHUNK, N), lambda b, c: (b, c, 0)),
            pl.BlockSpec((32, 128), lambda b, c: (0, 0)),
            pl.BlockSpec((32, 128), lambda b, c: (0, 0)),
            pl.BlockSpec((1, 512, 128), lambda b, c: (b, 0, 0)),
            pl.BlockSpec((1, 512, 128), lambda b, c: (b, 0, 0)),
        ],
        out_specs=pl.BlockSpec((1, 1), lambda b, c: (0, 0)),
        out_shape=jax.ShapeDtypeStruct((1, 1), jnp.float32),
        scratch_shapes=[
            pltpu.SMEM((1,), jnp.float32),
            pltpu.VMEM((8, N), jnp.float32),
            pltpu.VMEM((8, N), jnp.float32),
            pltpu.VMEM((8, N), jnp.float32),
            pltpu.VMEM((8, N), jnp.float32),
        ],
        compiler_params=pltpu.CompilerParams(
            dimension_semantics=("arbitrary", "arbitrary"),
        ),
    )(pred2, lab2, mask_logits, true_masks, posp, post, sal3, salt3)
    return out.reshape(())
